# baseline (speedup 1.0000x reference)
import functools

import jax
import jax.numpy as jnp
from jax import lax
from jax.experimental import pallas as pl
from jax.experimental.pallas import tpu as pltpu
from jax.experimental.pallas import tpu_sc as plsc

B = 1024
N = 1024
TOPO_DIM = 16
BIO_DIM = 112
K = 32
HIDDEN = 8


BB = 8


def _topk_body(topo_ref, wt_ref, idx_ref):
    i = pl.program_id(0)
    w = wt_ref[...]
    rows = []
    for b in range(BB):
        xb = topo_ref[b]
        rows.append(
            lax.dot_general(w, xb, (((1,), (1,)), ((), ())),
                            preferred_element_type=jnp.float32))
    sv = jnp.concatenate(rows, axis=0)
    iota_n = lax.broadcasted_iota(jnp.int32, (BB, N), 1)
    cols = []
    for _ in range(K):
        m = jnp.max(sv, axis=1, keepdims=True)
        am = jnp.min(jnp.where(sv == m, iota_n, N), axis=1, keepdims=True)
        cols.append(am)
        sv = jnp.where(iota_n == am, -jnp.inf, sv)
    idx = jnp.concatenate(cols, axis=1)
    gb = lax.broadcasted_iota(jnp.int32, (BB, K), 0) + i * BB
    idx_ref[...] = gb * N + idx


_topk_call = pl.pallas_call(
    _topk_body,
    grid=(B // BB,),
    in_specs=[
        pl.BlockSpec((BB, N, TOPO_DIM), lambda i: (i, 0, 0)),
        pl.BlockSpec((1, TOPO_DIM), lambda i: (0, 0)),
    ],
    out_specs=pl.BlockSpec((BB, K), lambda i: (i, 0)),
    out_shape=jax.ShapeDtypeStruct((B, K), jnp.int32),
)


NC = 2
NS = 16
NW = NC * NS
ROWS_PER_W = (B * K) // NW
CH = 128
NCH = ROWS_PER_W // CH

def _gather_body(topo_hbm, bio_hbm, idx_hbm, topo_out, bio_out,
                 idx0, idx1, t0, t1, b0, b1, st0, st1, sb0, sb1):
    wid = lax.axis_index("s") * NC + lax.axis_index("c")
    idxs = (idx0, idx1)
    ts = (t0, t1)
    bs = (b0, b1)
    sts = (st0, st1)
    sbs = (sb0, sb1)

    def issue(c):
        s = c % 2
        pltpu.sync_copy(idx_hbm.at[wid, c], idxs[s])
        ct = pltpu.async_copy(topo_hbm.at[idxs[s]], ts[s], sts[s])
        cb = pltpu.async_copy(bio_hbm.at[idxs[s]], bs[s], sbs[s])
        return ct, cb

    pend = {0: issue(0)}
    for c in range(NCH):
        if c + 1 < NCH:
            pend[c + 1] = issue(c + 1)
        ct, cb = pend.pop(c)
        ct.wait()
        cb.wait()
        s = c % 2
        base = wid * ROWS_PER_W + c * CH
        pltpu.sync_copy(ts[s], topo_out.at[pl.ds(base, CH)])
        pltpu.sync_copy(bs[s], bio_out.at[pl.ds(base, CH)])


@functools.cache
def _make_gather_call():
    mesh = plsc.VectorSubcoreMesh(core_axis_name="c", subcore_axis_name="s")
    return pl.kernel(
        _gather_body,
        mesh=mesh,
        compiler_params=pltpu.CompilerParams(use_tc_tiling_on_sc=False),
        out_type=[
            jax.ShapeDtypeStruct((B * K, TOPO_DIM), jnp.float32),
            jax.ShapeDtypeStruct((B * K, BIO_DIM), jnp.float32),
        ],
        scratch_types=[
            pltpu.VMEM((CH,), jnp.int32),
            pltpu.VMEM((CH,), jnp.int32),
            pltpu.VMEM((CH, TOPO_DIM), jnp.float32),
            pltpu.VMEM((CH, TOPO_DIM), jnp.float32),
            pltpu.VMEM((CH, BIO_DIM), jnp.float32),
            pltpu.VMEM((CH, BIO_DIM), jnp.float32),
            pltpu.SemaphoreType.DMA,
            pltpu.SemaphoreType.DMA,
            pltpu.SemaphoreType.DMA,
            pltpu.SemaphoreType.DMA,
        ],
    )


def _mlp_body(tg_ref, bg_ref, w1t_ref, w1b_ref, b1_ref, w2_ref, b2_ref,
              y_ref, score_ref, loss_ref):
    h = (jnp.dot(tg_ref[...], w1t_ref[...], preferred_element_type=jnp.float32)
         + jnp.dot(bg_ref[...], w1b_ref[...], preferred_element_type=jnp.float32)
         + b1_ref[...])
    h = jnp.maximum(h, 0.0)
    s = jnp.dot(h, w2_ref[...], preferred_element_type=jnp.float32) + b2_ref[...]
    score_ref[...] = s
    yf = y_ref[...]
    npos = jnp.sum(yf)
    pw = (jnp.float32(B) - npos) / npos
    ez = jnp.exp(-jnp.abs(s))
    log1pez = jnp.log(1.0 + ez)
    ls_pos = jnp.minimum(s, 0.0) - log1pez
    ls_neg = jnp.minimum(-s, 0.0) - log1pez
    l = -(pw * yf * ls_pos + (1.0 - yf) * ls_neg)
    loss_ref[...] = jnp.sum(l, axis=0, keepdims=True) * (1.0 / B)


_mlp_call = pl.pallas_call(
    _mlp_body,
    out_shape=[
        jax.ShapeDtypeStruct((B, 1), jnp.float32),
        jax.ShapeDtypeStruct((1, 1), jnp.float32),
    ],
)


def kernel(topo_feat, bio_feat, y, W_t, b_t, W1, b1, W2, b2):
    del b_t
    wt_row = W_t.reshape(1, TOPO_DIM)
    flat_idx = _topk_call(topo_feat, wt_row)
    idx3 = flat_idx.reshape(NW, NCH, CH)
    tg, bg = _make_gather_call()(topo_feat.reshape(B * N, TOPO_DIM),
                                 bio_feat.reshape(B * N, BIO_DIM), idx3)
    tg2 = tg.reshape(B, K * TOPO_DIM)
    bg2 = bg.reshape(B, K * BIO_DIM)
    w1r = W1.reshape(K, TOPO_DIM + BIO_DIM, HIDDEN)
    w1t = w1r[:, :TOPO_DIM, :].reshape(K * TOPO_DIM, HIDDEN)
    w1b = w1r[:, TOPO_DIM:, :].reshape(K * BIO_DIM, HIDDEN)
    score2, loss2 = _mlp_call(tg2, bg2, w1t, w1b, b1.reshape(1, HIDDEN),
                              W2, b2.reshape(1, 1),
                              y.astype(jnp.float32).reshape(B, 1))
    return (loss2.reshape(()), score2.reshape(B))


# speedup vs baseline: 6.5424x; 6.5424x over previous
import jax
import jax.numpy as jnp
from jax import lax
from jax.experimental import pallas as pl
from jax.experimental.pallas import tpu as pltpu

B = 1024
N = 1024
TOPO_DIM = 16
BIO_DIM = 112
D = TOPO_DIM + BIO_DIM
K = 32
HIDDEN = 8

BB = 32


def _sortpool_body(topo_ref, bio_ref, wt_ref, x_ref):
    w = wt_ref[...]
    rows = []
    for b in range(BB):
        rows.append(jnp.dot(w, topo_ref[b],
                            preferred_element_type=jnp.float32))
    sv = jnp.concatenate(rows, axis=0)
    iota_n = lax.broadcasted_iota(jnp.int32, (BB, N), 1)
    cols = []
    for _ in range(K):
        m = jnp.max(sv, axis=1, keepdims=True)
        am = jnp.min(jnp.where(sv == m, iota_n, N), axis=1, keepdims=True)
        cols.append(am)
        sv = jnp.where(iota_n == am, -jnp.inf, sv)
    idx = jnp.concatenate(cols, axis=1)
    idx_t = jnp.transpose(idx)
    iota_k = lax.broadcasted_iota(jnp.int32, (K, N), 1)
    parts = []
    for b in range(BB):
        mb = jnp.where(idx_t[:, b:b + 1] == iota_k, 1.0, 0.0)
        xt = lax.dot_general(mb, topo_ref[b], (((1,), (1,)), ((), ())),
                             preferred_element_type=jnp.float32)
        xb = lax.dot_general(mb, bio_ref[b], (((1,), (1,)), ((), ())),
                             preferred_element_type=jnp.float32)
        parts.append(jnp.concatenate([xt, xb], axis=1))
    x_ref[...] = jnp.concatenate(parts, axis=0)


_sortpool_call = pl.pallas_call(
    _sortpool_body,
    grid=(B // BB,),
    in_specs=[
        pl.BlockSpec((BB, TOPO_DIM, N), lambda i: (i, 0, 0)),
        pl.BlockSpec((BB, BIO_DIM, N), lambda i: (i, 0, 0)),
        pl.BlockSpec((1, TOPO_DIM), lambda i: (0, 0)),
    ],
    out_specs=pl.BlockSpec((BB * K, D), lambda i: (i, 0)),
    out_shape=jax.ShapeDtypeStruct((B * K, D), jnp.float32),
    compiler_params=pltpu.CompilerParams(vmem_limit_bytes=60 * 1024 * 1024),
)


def _mlp_body(x_ref, w1_ref, b1_ref, w2_ref, b2_ref, y_ref,
              score_ref, loss_ref):
    h = (jnp.dot(x_ref[...], w1_ref[...], preferred_element_type=jnp.float32)
         + b1_ref[...])
    h = jnp.maximum(h, 0.0)
    s = jnp.dot(h, w2_ref[...], preferred_element_type=jnp.float32) + b2_ref[...]
    score_ref[...] = s
    yf = y_ref[...]
    npos = jnp.sum(yf)
    pw = (jnp.float32(B) - npos) / npos
    ez = jnp.exp(-jnp.abs(s))
    log1pez = jnp.log(1.0 + ez)
    ls_pos = jnp.minimum(s, 0.0) - log1pez
    ls_neg = jnp.minimum(-s, 0.0) - log1pez
    l = -(pw * yf * ls_pos + (1.0 - yf) * ls_neg)
    loss_ref[...] = jnp.sum(l, axis=0, keepdims=True) * (1.0 / B)


_mlp_call = pl.pallas_call(
    _mlp_body,
    out_shape=[
        jax.ShapeDtypeStruct((B, 1), jnp.float32),
        jax.ShapeDtypeStruct((1, 1), jnp.float32),
    ],
)


def kernel(topo_feat, bio_feat, y, W_t, b_t, W1, b1, W2, b2):
    del b_t
    topo_t = jnp.transpose(topo_feat, (0, 2, 1))
    bio_t = jnp.transpose(bio_feat, (0, 2, 1))
    x = _sortpool_call(topo_t, bio_t, W_t.reshape(1, TOPO_DIM))
    x2 = x.reshape(B, K * D)
    score2, loss2 = _mlp_call(x2, W1, b1.reshape(1, HIDDEN), W2,
                              b2.reshape(1, 1),
                              y.astype(jnp.float32).reshape(B, 1))
    return (loss2.reshape(()), score2.reshape(B))


# speedup vs baseline: 10.4760x; 1.6012x over previous
import jax
import jax.numpy as jnp
from jax import lax
from jax.experimental import pallas as pl
from jax.experimental.pallas import tpu as pltpu

B = 1024
N = 1024
TOPO_DIM = 16
BIO_DIM = 112
D = TOPO_DIM + BIO_DIM
K = 32
HIDDEN = 8

BT = 128
BG = 32


def _topk_body(topo_ref, wt_ref, idxt_ref):
    w = wt_ref[...]
    rows = []
    for b in range(BT):
        rows.append(jnp.dot(w, topo_ref[b],
                            preferred_element_type=jnp.float32))
    sv = jnp.concatenate(rows, axis=0)
    iota_f = lax.broadcasted_iota(jnp.int32, (BT, N), 1).astype(jnp.float32)
    big = jnp.float32(N)
    cols = []
    for _ in range(K):
        m = jnp.max(sv, axis=1, keepdims=True)
        am = jnp.min(jnp.where(sv == m, iota_f, big), axis=1, keepdims=True)
        cols.append(am)
        sv = jnp.where(iota_f == am, -jnp.inf, sv)
    idx = jnp.concatenate(cols, axis=1).astype(jnp.int32)
    for j in range(BT // BG):
        idxt_ref[j] = jnp.transpose(idx[j * BG:(j + 1) * BG, :])


_topk_call = pl.pallas_call(
    _topk_body,
    grid=(B // BT,),
    in_specs=[
        pl.BlockSpec((BT, TOPO_DIM, N), lambda i: (i, 0, 0)),
        pl.BlockSpec((1, TOPO_DIM), lambda i: (0, 0)),
    ],
    out_specs=pl.BlockSpec((BT // BG, K, BG), lambda i: (i, 0, 0)),
    out_shape=jax.ShapeDtypeStruct((B // BG, K, BG), jnp.int32),
)


def _gather_body(topo_ref, bio_ref, idxt_ref, x_ref):
    iota_k = lax.broadcasted_iota(jnp.int32, (K, N), 1)
    idxt = idxt_ref[0]
    parts = []
    for b in range(BG):
        mb = jnp.where(idxt[:, b:b + 1] == iota_k, 1.0, 0.0)
        feat = jnp.concatenate([topo_ref[b], bio_ref[b]], axis=0)
        parts.append(
            lax.dot_general(mb, feat, (((1,), (1,)), ((), ())),
                            preferred_element_type=jnp.float32))
    x_ref[...] = jnp.concatenate(parts, axis=0)


_gather_call = pl.pallas_call(
    _gather_body,
    grid=(B // BG,),
    in_specs=[
        pl.BlockSpec((BG, TOPO_DIM, N), lambda i: (i, 0, 0)),
        pl.BlockSpec((BG, BIO_DIM, N), lambda i: (i, 0, 0)),
        pl.BlockSpec((1, K, BG), lambda i: (i, 0, 0)),
    ],
    out_specs=pl.BlockSpec((BG * K, D), lambda i: (i, 0)),
    out_shape=jax.ShapeDtypeStruct((B * K, D), jnp.float32),
    compiler_params=pltpu.CompilerParams(vmem_limit_bytes=60 * 1024 * 1024),
)


def _mlp_body(x_ref, w1_ref, b1_ref, w2_ref, b2_ref, y_ref,
              score_ref, loss_ref):
    h = (jnp.dot(x_ref[...], w1_ref[...], preferred_element_type=jnp.float32)
         + b1_ref[...])
    h = jnp.maximum(h, 0.0)
    s = jnp.dot(h, w2_ref[...], preferred_element_type=jnp.float32) + b2_ref[...]
    score_ref[...] = s
    yf = y_ref[...]
    npos = jnp.sum(yf)
    pw = (jnp.float32(B) - npos) / npos
    ez = jnp.exp(-jnp.abs(s))
    log1pez = jnp.log(1.0 + ez)
    ls_pos = jnp.minimum(s, 0.0) - log1pez
    ls_neg = jnp.minimum(-s, 0.0) - log1pez
    l = -(pw * yf * ls_pos + (1.0 - yf) * ls_neg)
    loss_ref[...] = jnp.sum(l, axis=0, keepdims=True) * (1.0 / B)


_mlp_call = pl.pallas_call(
    _mlp_body,
    out_shape=[
        jax.ShapeDtypeStruct((B, 1), jnp.float32),
        jax.ShapeDtypeStruct((1, 1), jnp.float32),
    ],
)


def kernel(topo_feat, bio_feat, y, W_t, b_t, W1, b1, W2, b2):
    del b_t
    topo_t = jnp.transpose(topo_feat, (0, 2, 1))
    bio_t = jnp.transpose(bio_feat, (0, 2, 1))
    idxt = _topk_call(topo_t, W_t.reshape(1, TOPO_DIM))
    x = _gather_call(topo_t, bio_t, idxt)
    x2 = x.reshape(B, K * D)
    score2, loss2 = _mlp_call(x2, W1, b1.reshape(1, HIDDEN), W2,
                              b2.reshape(1, 1),
                              y.astype(jnp.float32).reshape(B, 1))
    return (loss2.reshape(()), score2.reshape(B))


# speedup vs baseline: 10.8943x; 1.0399x over previous
import jax
import jax.numpy as jnp
from jax import lax
from jax.experimental import pallas as pl
from jax.experimental.pallas import tpu as pltpu

B = 1024
N = 1024
TOPO_DIM = 16
BIO_DIM = 112
D = TOPO_DIM + BIO_DIM
K = 32
HIDDEN = 8

BT = 128
BG = 32


SVB = 16


def _topk_body(topo_ref, wblk_ref, idxt_ref):
    wblk = wblk_ref[...]
    rows = []
    for b in range(0, BT, SVB):
        grp = topo_ref[b:b + SVB].reshape(SVB * TOPO_DIM, N)
        rows.append(jnp.dot(wblk, grp,
                            preferred_element_type=jnp.float32))
    sv = jnp.concatenate(rows, axis=0)
    iota_c = (lax.broadcasted_iota(jnp.int32, (N, 1), 0)
              .astype(jnp.float32))
    cols = []
    for _ in range(K):
        m = jnp.max(sv, axis=1, keepdims=True)
        pick = jnp.where(sv == m, 1.0, 0.0)
        cols.append(jnp.dot(pick, iota_c,
                            preferred_element_type=jnp.float32))
        sv = jnp.where(sv == m, -jnp.inf, sv)
    idx = jnp.concatenate(cols, axis=1).astype(jnp.int32)
    for j in range(BT // BG):
        idxt_ref[j] = jnp.transpose(idx[j * BG:(j + 1) * BG, :])


_topk_call = pl.pallas_call(
    _topk_body,
    grid=(B // BT,),
    in_specs=[
        pl.BlockSpec((BT, TOPO_DIM, N), lambda i: (i, 0, 0)),
        pl.BlockSpec((SVB, SVB * TOPO_DIM), lambda i: (0, 0)),
    ],
    out_specs=pl.BlockSpec((BT // BG, K, BG), lambda i: (i, 0, 0)),
    out_shape=jax.ShapeDtypeStruct((B // BG, K, BG), jnp.int32),
)


def _gather_body(topo_ref, bio_ref, idxt_ref, x_ref):
    iota_k = lax.broadcasted_iota(jnp.int32, (K, N), 1)
    idxt = idxt_ref[0]
    parts = []
    for b in range(BG):
        mb = jnp.where(idxt[:, b:b + 1] == iota_k, 1.0, 0.0)
        feat = jnp.concatenate([topo_ref[b], bio_ref[b]], axis=0)
        parts.append(
            lax.dot_general(mb, feat, (((1,), (1,)), ((), ())),
                            preferred_element_type=jnp.float32))
    x_ref[...] = jnp.concatenate(parts, axis=0)


_gather_call = pl.pallas_call(
    _gather_body,
    grid=(B // BG,),
    in_specs=[
        pl.BlockSpec((BG, TOPO_DIM, N), lambda i: (i, 0, 0)),
        pl.BlockSpec((BG, BIO_DIM, N), lambda i: (i, 0, 0)),
        pl.BlockSpec((1, K, BG), lambda i: (i, 0, 0)),
    ],
    out_specs=pl.BlockSpec((BG * K, D), lambda i: (i, 0)),
    out_shape=jax.ShapeDtypeStruct((B * K, D), jnp.float32),
    compiler_params=pltpu.CompilerParams(vmem_limit_bytes=60 * 1024 * 1024),
)


def _mlp_body(x_ref, w1_ref, b1_ref, w2_ref, b2_ref, y_ref,
              score_ref, loss_ref):
    h = (jnp.dot(x_ref[...], w1_ref[...], preferred_element_type=jnp.float32)
         + b1_ref[...])
    h = jnp.maximum(h, 0.0)
    s = jnp.dot(h, w2_ref[...], preferred_element_type=jnp.float32) + b2_ref[...]
    score_ref[...] = s
    yf = y_ref[...]
    npos = jnp.sum(yf)
    pw = (jnp.float32(B) - npos) / npos
    ez = jnp.exp(-jnp.abs(s))
    log1pez = jnp.log(1.0 + ez)
    ls_pos = jnp.minimum(s, 0.0) - log1pez
    ls_neg = jnp.minimum(-s, 0.0) - log1pez
    l = -(pw * yf * ls_pos + (1.0 - yf) * ls_neg)
    loss_ref[...] = jnp.sum(l, axis=0, keepdims=True) * (1.0 / B)


_mlp_call = pl.pallas_call(
    _mlp_body,
    out_shape=[
        jax.ShapeDtypeStruct((B, 1), jnp.float32),
        jax.ShapeDtypeStruct((1, 1), jnp.float32),
    ],
)


def kernel(topo_feat, bio_feat, y, W_t, b_t, W1, b1, W2, b2):
    del b_t
    topo_t = jnp.transpose(topo_feat, (0, 2, 1))
    bio_t = jnp.transpose(bio_feat, (0, 2, 1))
    wblk = jnp.kron(jnp.eye(SVB, dtype=jnp.float32),
                    W_t.reshape(1, TOPO_DIM))
    idxt = _topk_call(topo_t, wblk)
    x = _gather_call(topo_t, bio_t, idxt)
    x2 = x.reshape(B, K * D)
    score2, loss2 = _mlp_call(x2, W1, b1.reshape(1, HIDDEN), W2,
                              b2.reshape(1, 1),
                              y.astype(jnp.float32).reshape(B, 1))
    return (loss2.reshape(()), score2.reshape(B))
